# baseline (device time: 17324 ns/iter reference)
import jax
import jax.numpy as jnp
from jax import lax
from jax.experimental import pallas as pl
from jax.experimental.pallas import tpu as pltpu

N_DEV = 16
M = 256
N = 256
CH = M // N_DEV
HH = CH // 2


def kernel(x):
    def body(x_ref, out_ref, send_buf, rs_buf, ag_buf,
             send1, recv1, send2, recv2):
        me = lax.axis_index("i")

        barrier_sem = pltpu.get_barrier_semaphore()
        for k in range(1, N_DEV):
            p = lax.rem(me + k, N_DEV)
            pl.semaphore_signal(
                barrier_sem, inc=1,
                device_id=(p,), device_id_type=pl.DeviceIdType.MESH,
            )

        send_buf[...] = x_ref[0].astype(jnp.bfloat16)

        pl.semaphore_wait(barrier_sem, N_DEV - 1)

        sends = []

        for h in range(2):
            for k in range(1, N_DEV):
                p = lax.rem(me + k, N_DEV)
                rdma = pltpu.make_async_remote_copy(
                    src_ref=send_buf.at[pl.ds(p * CH + h * HH, HH), :],
                    dst_ref=rs_buf.at[h, me],
                    send_sem=send1.at[h, p],
                    recv_sem=recv1.at[h, me],
                    device_id=(p,),
                    device_id_type=pl.DeviceIdType.MESH,
                )
                rdma.start()
                sends.append(rdma)

        for h in range(2):
            acc = send_buf[pl.ds(me * CH + h * HH, HH), :].astype(jnp.float32)
            for k in range(1, N_DEV):
                s = lax.rem(me + k, N_DEV)
                pltpu.make_async_remote_copy(
                    src_ref=send_buf.at[pl.ds(0, HH), :],
                    dst_ref=rs_buf.at[h, s],
                    send_sem=send1.at[h, s],
                    recv_sem=recv1.at[h, s],
                    device_id=(s,),
                    device_id_type=pl.DeviceIdType.MESH,
                ).wait_recv()
                acc = acc + rs_buf[h, s].astype(jnp.float32)

            ag_buf[h] = acc.astype(jnp.bfloat16)
            out_ref[pl.ds(me * CH + h * HH, HH), :] = ag_buf[h]

            for k in range(1, N_DEV):
                p = lax.rem(me + k, N_DEV)
                rdma = pltpu.make_async_remote_copy(
                    src_ref=ag_buf.at[h],
                    dst_ref=out_ref.at[pl.ds(me * CH + h * HH, HH), :],
                    send_sem=send2.at[h, p],
                    recv_sem=recv2.at[h, me],
                    device_id=(p,),
                    device_id_type=pl.DeviceIdType.MESH,
                )
                rdma.start()
                sends.append(rdma)

        for h in range(2):
            for k in range(1, N_DEV):
                s = lax.rem(me + k, N_DEV)
                pltpu.make_async_remote_copy(
                    src_ref=ag_buf.at[h],
                    dst_ref=out_ref.at[pl.ds(s * CH + h * HH, HH), :],
                    send_sem=send2.at[h, s],
                    recv_sem=recv2.at[h, s],
                    device_id=(s,),
                    device_id_type=pl.DeviceIdType.MESH,
                ).wait_recv()

        for rdma in sends:
            rdma.wait_send()

    return pl.pallas_call(
        body,
        out_shape=jax.ShapeDtypeStruct((M, N), jnp.bfloat16),
        in_specs=[pl.BlockSpec(memory_space=pltpu.VMEM)],
        out_specs=pl.BlockSpec(memory_space=pltpu.VMEM),
        scratch_shapes=[
            pltpu.VMEM((M, N), jnp.bfloat16),
            pltpu.VMEM((2, N_DEV, HH, N), jnp.bfloat16),
            pltpu.VMEM((2, HH, N), jnp.bfloat16),
            pltpu.SemaphoreType.DMA((2, N_DEV)),
            pltpu.SemaphoreType.DMA((2, N_DEV)),
            pltpu.SemaphoreType.DMA((2, N_DEV)),
            pltpu.SemaphoreType.DMA((2, N_DEV)),
        ],
        compiler_params=pltpu.CompilerParams(collective_id=0),
    )(x)
